# Initial kernel scaffold; baseline (speedup 1.0000x reference)
import jax
import jax.numpy as jnp
from jax.experimental import pallas as pl


def kernel(post_activation_sincos, has_rotation, sincos, object_idxs, img_idxs, head_idxs, grid_y_idxs, grid_x_idxs):
    raise NotImplementedError("write your pallas kernel here")


# speedup vs baseline: 5.1157x; 5.1157x over previous
import functools

import jax
import jax.numpy as jnp
from jax import lax
from jax.experimental import pallas as pl
from jax.experimental.pallas import tpu as pltpu
from jax.experimental.pallas import tpu_sc as plsc

B, H, GY, GX = 16, 4, 64, 64
NUM_OBJ = 5000
NUM_ASSIGN = 20000

NC, NS, L = 2, 16, 16
NW = NC * NS
CHUNK = 640
APAD = NW * CHUNK
NV = CHUNK // L


def _sc_body(pred_hbm, hr_hbm, sc_hbm, obj_hbm, img_hbm, head_hbm, gy_hbm,
             gx_hbm, out_hbm,
             obj_v, img_v, head_v, gy_v, gx_v,
             ip0_v, ip1_v, it0_v, it1_v,
             hr_v, t0_v, t1_v, p0_v, p1_v,
             acc_v, sem):
    cid = lax.axis_index("c")
    sid = lax.axis_index("s")
    wid = sid * NC + cid
    base = wid * CHUNK

    pltpu.sync_copy(obj_hbm.at[pl.ds(base, CHUNK)], obj_v)
    pltpu.sync_copy(img_hbm.at[pl.ds(base, CHUNK)], img_v)
    pltpu.sync_copy(head_hbm.at[pl.ds(base, CHUNK)], head_v)
    pltpu.sync_copy(gy_hbm.at[pl.ds(base, CHUNK)], gy_v)
    pltpu.sync_copy(gx_hbm.at[pl.ds(base, CHUNK)], gx_v)

    for i in range(NV):
        sl = pl.ds(i * L, L)
        obj = obj_v[sl]
        flat = ((img_v[sl] * H + head_v[sl]) * 2) * (GY * GX) \
            + gy_v[sl] * GX + gx_v[sl]
        ip0_v[sl] = flat
        ip1_v[sl] = flat + GY * GX
        it0_v[sl] = obj * 2
        it1_v[sl] = obj * 2 + 1

    cp0 = pltpu.async_copy(hr_hbm.at[obj_v], hr_v, sem)
    cp1 = pltpu.async_copy(sc_hbm.at[it0_v], t0_v, sem)
    cp2 = pltpu.async_copy(sc_hbm.at[it1_v], t1_v, sem)
    cp3 = pltpu.async_copy(pred_hbm.at[ip0_v], p0_v, sem)
    cp4 = pltpu.async_copy(pred_hbm.at[ip1_v], p1_v, sem)
    cp0.wait()
    cp1.wait()
    cp2.wait()
    cp3.wait()
    cp4.wait()

    iota = lax.iota(jnp.int32, L)
    acc = jnp.zeros((L,), jnp.float32)
    for i in range(NV):
        sl = pl.ds(i * L, L)
        pos = base + i * L + iota
        m = (hr_v[sl] != 0) & (pos < NUM_ASSIGN)
        d0 = t0_v[sl] - p0_v[sl]
        d1 = t1_v[sl] - p1_v[sl]
        acc = acc + jnp.where(m, d0 * d0 + d1 * d1, 0.0)
    acc_v[:] = acc
    pltpu.sync_copy(acc_v, out_hbm.at[wid])


@jax.jit
def _sc_loss(pred_flat, has_rotation, sc_flat, obj, img, head, gy, gx):
    mesh = plsc.VectorSubcoreMesh(core_axis_name="c", subcore_axis_name="s")
    run = functools.partial(
        pl.kernel,
        mesh=mesh,
        out_type=jax.ShapeDtypeStruct((NW, L), jnp.float32),
        scratch_types=[
            pltpu.VMEM((CHUNK,), jnp.int32),
            pltpu.VMEM((CHUNK,), jnp.int32),
            pltpu.VMEM((CHUNK,), jnp.int32),
            pltpu.VMEM((CHUNK,), jnp.int32),
            pltpu.VMEM((CHUNK,), jnp.int32),
            pltpu.VMEM((CHUNK,), jnp.int32),
            pltpu.VMEM((CHUNK,), jnp.int32),
            pltpu.VMEM((CHUNK,), jnp.int32),
            pltpu.VMEM((CHUNK,), jnp.int32),
            pltpu.VMEM((CHUNK,), jnp.int32),
            pltpu.VMEM((CHUNK,), jnp.float32),
            pltpu.VMEM((CHUNK,), jnp.float32),
            pltpu.VMEM((CHUNK,), jnp.float32),
            pltpu.VMEM((CHUNK,), jnp.float32),
            pltpu.VMEM((L,), jnp.float32),
            pltpu.SemaphoreType.DMA,
        ],
    )(_sc_body)
    out = run(pred_flat, has_rotation, sc_flat, obj, img, head, gy, gx)
    return jnp.sum(out)


def kernel(post_activation_sincos, has_rotation, sincos, object_idxs,
           img_idxs, head_idxs, grid_y_idxs, grid_x_idxs):
    pred_flat = post_activation_sincos.reshape(-1)
    sc_flat = sincos.reshape(-1)
    pad = APAD - NUM_ASSIGN
    obj = jnp.pad(object_idxs, (0, pad))
    img = jnp.pad(img_idxs, (0, pad))
    head = jnp.pad(head_idxs, (0, pad))
    gy = jnp.pad(grid_y_idxs, (0, pad))
    gx = jnp.pad(grid_x_idxs, (0, pad))
    return _sc_loss(pred_flat, has_rotation.astype(jnp.int32), sc_flat,
                    obj, img, head, gy, gx)


# speedup vs baseline: 7.4067x; 1.4478x over previous
import functools

import jax
import jax.numpy as jnp
from jax import lax
from jax.experimental import pallas as pl
from jax.experimental.pallas import tpu as pltpu
from jax.experimental.pallas import tpu_sc as plsc

B, H, GY, GX = 16, 4, 64, 64
NUM_OBJ = 5000
NUM_ASSIGN = 20000

NC, NS, L = 2, 16, 16
NW = NC * NS
CHUNK = 640
NV = CHUNK // L


def _sc_body(pred_hbm, hr_hbm, sc_hbm, obj_hbm, img_hbm, head_hbm, gy_hbm,
             gx_hbm, out_hbm,
             hr_tab, sc_tab,
             obj_v, img_v, head_v, gy_v, gx_v,
             ip0_v, ip1_v, p0_v, p1_v,
             acc_v, sem_idx, sem_tab):
    cid = lax.axis_index("c")
    sid = lax.axis_index("s")
    wid = sid * NC + cid
    own = wid * CHUNK
    base = jnp.minimum(own, NUM_ASSIGN - CHUNK)

    ct0 = pltpu.async_copy(hr_hbm, hr_tab, sem_tab)
    ct1 = pltpu.async_copy(sc_hbm, sc_tab, sem_tab)
    ci0 = pltpu.async_copy(obj_hbm.at[pl.ds(base, CHUNK)], obj_v, sem_idx)
    ci1 = pltpu.async_copy(img_hbm.at[pl.ds(base, CHUNK)], img_v, sem_idx)
    ci2 = pltpu.async_copy(head_hbm.at[pl.ds(base, CHUNK)], head_v, sem_idx)
    ci3 = pltpu.async_copy(gy_hbm.at[pl.ds(base, CHUNK)], gy_v, sem_idx)
    ci4 = pltpu.async_copy(gx_hbm.at[pl.ds(base, CHUNK)], gx_v, sem_idx)
    ci0.wait()
    ci1.wait()
    ci2.wait()
    ci3.wait()
    ci4.wait()

    for i in range(NV):
        sl = pl.ds(i * L, L)
        flat = ((img_v[sl] * H + head_v[sl]) * 2) * (GY * GX) \
            + gy_v[sl] * GX + gx_v[sl]
        ip0_v[sl] = flat
        ip1_v[sl] = flat + GY * GX

    cp0 = pltpu.async_copy(pred_hbm.at[ip0_v], p0_v, sem_tab)
    cp1 = pltpu.async_copy(pred_hbm.at[ip1_v], p1_v, sem_tab)
    ct0.wait()
    ct1.wait()
    cp0.wait()
    cp1.wait()

    iota = lax.iota(jnp.int32, L)
    acc = jnp.zeros((L,), jnp.float32)
    for i in range(NV):
        sl = pl.ds(i * L, L)
        obj = obj_v[sl]
        hr = plsc.load_gather(hr_tab, [obj])
        t0 = plsc.load_gather(sc_tab, [obj * 2])
        t1 = plsc.load_gather(sc_tab, [obj * 2 + 1])
        pos = base + i * L + iota
        m = (hr != 0) & (pos >= own)
        d0 = t0 - p0_v[sl]
        d1 = t1 - p1_v[sl]
        acc = acc + jnp.where(m, d0 * d0 + d1 * d1, 0.0)
    acc_v[:] = acc
    pltpu.sync_copy(acc_v, out_hbm.at[wid])


@jax.jit
def _sc_loss(pred_flat, has_rotation, sc_flat, obj, img, head, gy, gx):
    mesh = plsc.VectorSubcoreMesh(core_axis_name="c", subcore_axis_name="s")
    run = functools.partial(
        pl.kernel,
        mesh=mesh,
        compiler_params=pltpu.CompilerParams(needs_layout_passes=False),
        out_type=jax.ShapeDtypeStruct((NW, L), jnp.float32),
        scratch_types=[
            pltpu.VMEM((NUM_OBJ,), jnp.int32),
            pltpu.VMEM((2 * NUM_OBJ,), jnp.float32),
            pltpu.VMEM((CHUNK,), jnp.int32),
            pltpu.VMEM((CHUNK,), jnp.int32),
            pltpu.VMEM((CHUNK,), jnp.int32),
            pltpu.VMEM((CHUNK,), jnp.int32),
            pltpu.VMEM((CHUNK,), jnp.int32),
            pltpu.VMEM((CHUNK,), jnp.int32),
            pltpu.VMEM((CHUNK,), jnp.int32),
            pltpu.VMEM((CHUNK,), jnp.float32),
            pltpu.VMEM((CHUNK,), jnp.float32),
            pltpu.VMEM((L,), jnp.float32),
            pltpu.SemaphoreType.DMA,
            pltpu.SemaphoreType.DMA,
        ],
    )(_sc_body)
    out = run(pred_flat, has_rotation, sc_flat, obj, img, head, gy, gx)
    return jnp.sum(out)


def kernel(post_activation_sincos, has_rotation, sincos, object_idxs,
           img_idxs, head_idxs, grid_y_idxs, grid_x_idxs):
    pred_flat = post_activation_sincos.reshape(-1)
    sc_flat = sincos.reshape(-1)
    return _sc_loss(pred_flat, has_rotation.astype(jnp.int32), sc_flat,
                    object_idxs, img_idxs, head_idxs, grid_y_idxs,
                    grid_x_idxs)


# speedup vs baseline: 7.5191x; 1.0152x over previous
import functools

import jax
import jax.numpy as jnp
from jax import lax
from jax.experimental import pallas as pl
from jax.experimental.pallas import tpu as pltpu
from jax.experimental.pallas import tpu_sc as plsc

B, H, GY, GX = 16, 4, 64, 64
NUM_OBJ = 5000
NUM_ASSIGN = 20000

NC, NS, L = 2, 16, 16
NW = NC * NS
CHUNK = 640
NV = CHUNK // L


def _sc_body(pred_hbm, hr_hbm, sc_hbm, obj_hbm, img_hbm, head_hbm, gy_hbm,
             gx_hbm, out_hbm,
             hr_tab, sc_tab,
             obj_v, img_v, head_v, gy_v, gx_v,
             ip0_v, ip1_v, p0_v, p1_v,
             acc_v, sem_idx, sem_tab):
    cid = lax.axis_index("c")
    sid = lax.axis_index("s")
    wid = sid * NC + cid
    own = wid * CHUNK
    base = jnp.minimum(own, NUM_ASSIGN - CHUNK)

    ct0 = pltpu.async_copy(hr_hbm, hr_tab, sem_tab)
    ct1 = pltpu.async_copy(sc_hbm, sc_tab, sem_tab)
    ci0 = pltpu.async_copy(obj_hbm.at[pl.ds(base, CHUNK)], obj_v, sem_idx)
    ci1 = pltpu.async_copy(img_hbm.at[pl.ds(base, CHUNK)], img_v, sem_idx)
    ci2 = pltpu.async_copy(head_hbm.at[pl.ds(base, CHUNK)], head_v, sem_idx)
    ci3 = pltpu.async_copy(gy_hbm.at[pl.ds(base, CHUNK)], gy_v, sem_idx)
    ci4 = pltpu.async_copy(gx_hbm.at[pl.ds(base, CHUNK)], gx_v, sem_idx)
    ci0.wait()
    ci1.wait()
    ci2.wait()
    ci3.wait()
    ci4.wait()

    def idx_body(i, carry):
        sl = pl.ds(i * L, L)
        flat = ((img_v[sl] * H + head_v[sl]) * 2) * (GY * GX) \
            + gy_v[sl] * GX + gx_v[sl]
        ip0_v[sl] = flat
        ip1_v[sl] = flat + GY * GX
        return carry

    lax.fori_loop(0, NV, idx_body, 0)

    cp0 = pltpu.async_copy(pred_hbm.at[ip0_v], p0_v, sem_tab)
    cp1 = pltpu.async_copy(pred_hbm.at[ip1_v], p1_v, sem_tab)
    ct0.wait()
    ct1.wait()
    cp0.wait()
    cp1.wait()

    iota = lax.iota(jnp.int32, L)

    def red_body(i, acc):
        sl = pl.ds(i * L, L)
        obj = obj_v[sl]
        hr = plsc.load_gather(hr_tab, [obj])
        t0 = plsc.load_gather(sc_tab, [obj * 2])
        t1 = plsc.load_gather(sc_tab, [obj * 2 + 1])
        pos = base + i * L + iota
        m = (hr != 0) & (pos >= own)
        d0 = t0 - p0_v[sl]
        d1 = t1 - p1_v[sl]
        return acc + jnp.where(m, d0 * d0 + d1 * d1, 0.0)

    acc = lax.fori_loop(0, NV, red_body, jnp.zeros((L,), jnp.float32))
    acc_v[:] = acc
    pltpu.sync_copy(acc_v, out_hbm.at[wid])


@jax.jit
def _sc_loss(pred_flat, has_rotation, sc_flat, obj, img, head, gy, gx):
    mesh = plsc.VectorSubcoreMesh(core_axis_name="c", subcore_axis_name="s")
    run = functools.partial(
        pl.kernel,
        mesh=mesh,
        compiler_params=pltpu.CompilerParams(needs_layout_passes=False),
        out_type=jax.ShapeDtypeStruct((NW, L), jnp.float32),
        scratch_types=[
            pltpu.VMEM((NUM_OBJ,), jnp.int32),
            pltpu.VMEM((2 * NUM_OBJ,), jnp.float32),
            pltpu.VMEM((CHUNK,), jnp.int32),
            pltpu.VMEM((CHUNK,), jnp.int32),
            pltpu.VMEM((CHUNK,), jnp.int32),
            pltpu.VMEM((CHUNK,), jnp.int32),
            pltpu.VMEM((CHUNK,), jnp.int32),
            pltpu.VMEM((CHUNK,), jnp.int32),
            pltpu.VMEM((CHUNK,), jnp.int32),
            pltpu.VMEM((CHUNK,), jnp.float32),
            pltpu.VMEM((CHUNK,), jnp.float32),
            pltpu.VMEM((L,), jnp.float32),
            pltpu.SemaphoreType.DMA,
            pltpu.SemaphoreType.DMA,
        ],
    )(_sc_body)
    out = run(pred_flat, has_rotation, sc_flat, obj, img, head, gy, gx)
    return jnp.sum(out)


def kernel(post_activation_sincos, has_rotation, sincos, object_idxs,
           img_idxs, head_idxs, grid_y_idxs, grid_x_idxs):
    pred_flat = post_activation_sincos.reshape(-1)
    sc_flat = sincos.reshape(-1)
    return _sc_loss(pred_flat, has_rotation.astype(jnp.int32), sc_flat,
                    object_idxs, img_idxs, head_idxs, grid_y_idxs,
                    grid_x_idxs)


# speedup vs baseline: 7.5530x; 1.0045x over previous
import functools

import jax
import jax.numpy as jnp
from jax import lax
from jax.experimental import pallas as pl
from jax.experimental.pallas import tpu as pltpu
from jax.experimental.pallas import tpu_sc as plsc

B, H, GY, GX = 16, 4, 64, 64
NUM_OBJ = 5000
NUM_ASSIGN = 20000

NC, NS, L = 2, 16, 16
NW = NC * NS
CHUNK = 640
NV = CHUNK // L


def _sc_body(pred_hbm, hr_hbm, sc_hbm, obj_hbm, img_hbm, head_hbm, gy_hbm,
             gx_hbm, out_hbm,
             hr_tab, sc_tab,
             obj_v, img_v, head_v, gy_v, gx_v,
             ip0_v, ip1_v, p0_v, p1_v,
             acc_v, sem_idx, sem_tab):
    cid = lax.axis_index("c")
    sid = lax.axis_index("s")
    wid = sid * NC + cid
    own = wid * CHUNK
    base = jnp.minimum(own, NUM_ASSIGN - CHUNK)

    ct0 = pltpu.async_copy(hr_hbm, hr_tab, sem_tab)
    ct1 = pltpu.async_copy(sc_hbm, sc_tab, sem_tab)
    ci0 = pltpu.async_copy(obj_hbm.at[pl.ds(base, CHUNK)], obj_v, sem_idx)
    ci1 = pltpu.async_copy(img_hbm.at[pl.ds(base, CHUNK)], img_v, sem_idx)
    ci2 = pltpu.async_copy(head_hbm.at[pl.ds(base, CHUNK)], head_v, sem_idx)
    ci3 = pltpu.async_copy(gy_hbm.at[pl.ds(base, CHUNK)], gy_v, sem_idx)
    ci4 = pltpu.async_copy(gx_hbm.at[pl.ds(base, CHUNK)], gx_v, sem_idx)
    ci0.wait()
    ci1.wait()
    ci2.wait()
    ci3.wait()
    ci4.wait()

    def idx_body(i, carry):
        sl = pl.ds(i * L, L)
        flat = ((img_v[sl] * H + head_v[sl]) * 2) * (GY * GX) \
            + gy_v[sl] * GX + gx_v[sl]
        ip0_v[sl] = flat
        ip1_v[sl] = flat + GY * GX
        return carry

    lax.fori_loop(0, NV, idx_body, 0)

    cp0 = pltpu.async_copy(pred_hbm.at[ip0_v], p0_v, sem_tab)
    cp1 = pltpu.async_copy(pred_hbm.at[ip1_v], p1_v, sem_tab)
    ct0.wait()
    ct1.wait()
    cp0.wait()
    cp1.wait()

    iota = lax.iota(jnp.int32, L)

    def red_body(i, acc):
        sl = pl.ds(i * L, L)
        obj = obj_v[sl]
        hr = plsc.load_gather(hr_tab, [obj])
        t0 = plsc.load_gather(sc_tab, [obj * 2])
        t1 = plsc.load_gather(sc_tab, [obj * 2 + 1])
        pos = base + i * L + iota
        m = (hr != 0) & (pos >= own)
        d0 = t0 - p0_v[sl]
        d1 = t1 - p1_v[sl]
        return acc + jnp.where(m, d0 * d0 + d1 * d1, 0.0)

    acc = lax.fori_loop(0, NV, red_body, jnp.zeros((L,), jnp.float32))
    acc_v[:] = acc
    pltpu.sync_copy(acc_v, out_hbm.at[wid])


@jax.jit
def _sc_loss(pred_flat, has_rotation, sc_flat, obj, img, head, gy, gx):
    mesh = plsc.VectorSubcoreMesh(core_axis_name="c", subcore_axis_name="s")
    run = functools.partial(
        pl.kernel,
        mesh=mesh,
        compiler_params=pltpu.CompilerParams(needs_layout_passes=False,
                                             skip_device_barrier=True),
        out_type=jax.ShapeDtypeStruct((NW, L), jnp.float32),
        scratch_types=[
            pltpu.VMEM((NUM_OBJ,), jnp.int32),
            pltpu.VMEM((2 * NUM_OBJ,), jnp.float32),
            pltpu.VMEM((CHUNK,), jnp.int32),
            pltpu.VMEM((CHUNK,), jnp.int32),
            pltpu.VMEM((CHUNK,), jnp.int32),
            pltpu.VMEM((CHUNK,), jnp.int32),
            pltpu.VMEM((CHUNK,), jnp.int32),
            pltpu.VMEM((CHUNK,), jnp.int32),
            pltpu.VMEM((CHUNK,), jnp.int32),
            pltpu.VMEM((CHUNK,), jnp.float32),
            pltpu.VMEM((CHUNK,), jnp.float32),
            pltpu.VMEM((L,), jnp.float32),
            pltpu.SemaphoreType.DMA,
            pltpu.SemaphoreType.DMA,
        ],
    )(_sc_body)
    out = run(pred_flat, has_rotation, sc_flat, obj, img, head, gy, gx)
    return jnp.sum(out)


def kernel(post_activation_sincos, has_rotation, sincos, object_idxs,
           img_idxs, head_idxs, grid_y_idxs, grid_x_idxs):
    pred_flat = post_activation_sincos.reshape(-1)
    sc_flat = sincos.reshape(-1)
    return _sc_loss(pred_flat, has_rotation.astype(jnp.int32), sc_flat,
                    object_idxs, img_idxs, head_idxs, grid_y_idxs,
                    grid_x_idxs)
